# Initial kernel scaffold; baseline (speedup 1.0000x reference)
import jax
import jax.numpy as jnp
from jax.experimental import pallas as pl


def kernel(seq_len, table):
    raise NotImplementedError("write your pallas kernel here")


# speedup vs baseline: 1.5457x; 1.5457x over previous
import functools

import jax
import jax.numpy as jnp
from jax import lax
from jax.experimental import pallas as pl
from jax.experimental.pallas import tpu as pltpu
from jax.experimental.pallas import tpu_sc as plsc

_B = 8192
_D = 2048
_NC = 2
_NS = 16
_NW = _NC * _NS
_ROWS_PER_W = _B // _NW
_CH = 16
_NCHUNK = _ROWS_PER_W // _CH


def _pos_gather_body(lim_hbm, table_hbm, out_hbm,
                     lim_v, idx0, idx1, rows0, rows1, sem0, sem1):
    wid = lax.axis_index("s") * _NC + lax.axis_index("c")
    base = wid * _ROWS_PER_W

    pltpu.sync_copy(lim_hbm, lim_v)
    limit = lim_v[...]
    lanes = lax.iota(jnp.int32, _CH)

    idx_refs = (idx0, idx1)
    row_refs = (rows0, rows1)
    sems = (sem0, sem1)

    def fill_and_fire(g):
        slot = g % 2
        row0 = base + g * _CH
        idx_refs[slot][...] = jnp.minimum(row0 + lanes, limit)
        return pltpu.async_copy(table_hbm.at[idx_refs[slot]],
                                row_refs[slot], sems[slot])

    copies = [None, None]
    copies[0] = fill_and_fire(0)
    for g in range(_NCHUNK):
        nxt = g + 1
        if nxt < _NCHUNK:
            copies[nxt % 2] = fill_and_fire(nxt)
        copies[g % 2].wait()
        pltpu.sync_copy(row_refs[g % 2],
                        out_hbm.at[pl.ds(base + g * _CH, _CH)])


_pos_gather = functools.partial(
    pl.kernel,
    out_type=jax.ShapeDtypeStruct((_B, _D), jnp.float32),
    mesh=plsc.VectorSubcoreMesh(core_axis_name="c", subcore_axis_name="s"),
    scratch_types=[
        pltpu.VMEM((_CH,), jnp.int32),
        pltpu.VMEM((_CH,), jnp.int32),
        pltpu.VMEM((_CH,), jnp.int32),
        pltpu.VMEM((_CH, _D), jnp.float32),
        pltpu.VMEM((_CH, _D), jnp.float32),
        pltpu.SemaphoreType.DMA,
        pltpu.SemaphoreType.DMA,
    ],
)(_pos_gather_body)


def kernel(seq_len, table):
    lim = jnp.full((_CH,), jnp.asarray(seq_len, jnp.int32) - 1, jnp.int32)
    out = _pos_gather(lim, table)
    return out[None]


# speedup vs baseline: 1.5589x; 1.0085x over previous
import functools

import jax
import jax.numpy as jnp
from jax import lax
from jax.experimental import pallas as pl
from jax.experimental.pallas import tpu as pltpu
from jax.experimental.pallas import tpu_sc as plsc

_B = 8192
_D = 2048
_NC = 2
_NS = 16
_NW = _NC * _NS
_ROWS_PER_W = _B // _NW
_CH = 16
_NCHUNK = _ROWS_PER_W // _CH


_NBUF = 3


def _pos_gather_body(lim_hbm, table_hbm, out_hbm, lim_v,
                     idx0, idx1, idx2, rows0, rows1, rows2,
                     gsem0, gsem1, gsem2, ssem0, ssem1, ssem2):
    wid = lax.axis_index("s") * _NC + lax.axis_index("c")
    base = wid * _ROWS_PER_W

    pltpu.sync_copy(lim_hbm, lim_v)
    limit = lim_v[...]
    lanes = lax.iota(jnp.int32, _CH)

    idx_refs = (idx0, idx1, idx2)
    row_refs = (rows0, rows1, rows2)
    gsems = (gsem0, gsem1, gsem2)
    ssems = (ssem0, ssem1, ssem2)

    def fire_gather(g):
        slot = g % _NBUF
        row0 = base + g * _CH
        idx_refs[slot][...] = jnp.minimum(row0 + lanes, limit)
        return pltpu.async_copy(table_hbm.at[idx_refs[slot]],
                                row_refs[slot], gsems[slot])

    gathers = [None] * _NBUF
    stores = [None] * _NBUF
    for b in range(_NBUF):
        gathers[b] = fire_gather(b)
    for g in range(_NCHUNK):
        slot = g % _NBUF
        gathers[slot].wait()
        stores[slot] = pltpu.async_copy(
            row_refs[slot], out_hbm.at[pl.ds(base + g * _CH, _CH)],
            ssems[slot])
        nxt = g + _NBUF
        if nxt < _NCHUNK:
            stores[slot].wait()
            gathers[slot] = fire_gather(nxt)
        elif g >= _NCHUNK - _NBUF:
            stores[slot].wait()


_pos_gather = functools.partial(
    pl.kernel,
    out_type=jax.ShapeDtypeStruct((_B, _D), jnp.float32),
    mesh=plsc.VectorSubcoreMesh(core_axis_name="c", subcore_axis_name="s"),
    scratch_types=[
        pltpu.VMEM((_CH,), jnp.int32),
        pltpu.VMEM((_CH,), jnp.int32),
        pltpu.VMEM((_CH,), jnp.int32),
        pltpu.VMEM((_CH,), jnp.int32),
        pltpu.VMEM((_CH, _D), jnp.float32),
        pltpu.VMEM((_CH, _D), jnp.float32),
        pltpu.VMEM((_CH, _D), jnp.float32),
        pltpu.SemaphoreType.DMA,
        pltpu.SemaphoreType.DMA,
        pltpu.SemaphoreType.DMA,
        pltpu.SemaphoreType.DMA,
        pltpu.SemaphoreType.DMA,
        pltpu.SemaphoreType.DMA,
    ],
)(_pos_gather_body)


def kernel(seq_len, table):
    lim = jnp.full((_CH,), jnp.asarray(seq_len, jnp.int32) - 1, jnp.int32)
    out = _pos_gather(lim, table)
    return out[None]
